# baseline (speedup 1.0000x reference)
import functools

import jax
import jax.numpy as jnp
from jax import lax
from jax.experimental import pallas as pl
from jax.experimental.pallas import tpu as pltpu
from jax.experimental.pallas import tpu_sc as plsc

V = 1000000
D = 64
R = 16
B = 16384
L = 20

NC, NS = 2, 16
NW = NC * NS
N = B * L
NB = 256
CPL = B // NB
CPW = L * CPL // NW


def _adap_body(idx0_ref, wd_ref, bd_ref, wu_ref, bu_ref, table_ref,
               out_ref, emb_ref, sem):
    for i in range(L):
        pltpu.make_async_copy(
            table_ref.at[pl.ds(idx0_ref[i], 1)], emb_ref.at[pl.ds(i, 1)], sem
        ).start()
    for i in range(L):
        pltpu.make_async_copy(
            table_ref.at[pl.ds(idx0_ref[i], 1)], emb_ref.at[pl.ds(i, 1)], sem
        ).wait()
    h = emb_ref[...]
    mid = jnp.maximum(
        jnp.dot(h, wd_ref[...], preferred_element_type=jnp.float32)
        + bd_ref[...], 0.0)
    out_ref[...] = (h
                    + jnp.dot(mid, wu_ref[...],
                              preferred_element_type=jnp.float32)
                    + bu_ref[...])


_adapter_call = pl.pallas_call(
    _adap_body,
    out_shape=jax.ShapeDtypeStruct((L, D), jnp.float32),
    in_specs=[
        pl.BlockSpec(memory_space=pltpu.SMEM),
        pl.BlockSpec(memory_space=pltpu.VMEM),
        pl.BlockSpec(memory_space=pltpu.VMEM),
        pl.BlockSpec(memory_space=pltpu.VMEM),
        pl.BlockSpec(memory_space=pltpu.VMEM),
        pl.BlockSpec(memory_space=pltpu.MemorySpace.HBM),
    ],
    out_specs=pl.BlockSpec(memory_space=pltpu.VMEM),
    scratch_shapes=[pltpu.VMEM((L, D), jnp.float32), pltpu.SemaphoreType.DMA],
)


TK = 1024
TG = (V + TK - 1) // TK


def _tp_body(tt_ref, out_ref):
    out_ref[:, :D] = jnp.transpose(tt_ref[...], (1, 0))


_tp_call = pl.pallas_call(
    _tp_body,
    grid=(TG,),
    in_specs=[pl.BlockSpec((D, TK), lambda j: (0, j))],
    out_specs=pl.BlockSpec((TK, 128), lambda j: (j, 0)),
    out_shape=jax.ShapeDtypeStruct((V, 128), jnp.float32),
)


def _sc_body(table2, idxf, adap, out,
             idx_v, idx2_v, adap_v, spl_v, rows_v, tr_v,
             is0, is1, gs0, gs1, ss0, ss1):
    wid = lax.axis_index("s") * NC + lax.axis_index("c")
    cbase = wid * CPW
    l0 = cbase // CPL

    pltpu.sync_copy(adap, adap_v)

    one = jnp.int32(1)
    iota = lax.iota(jnp.int32, 16)

    for li in range(2):
        l = jnp.minimum(l0 + li, L - 1)
        for d in range(D):
            base = l * D + (d // 16) * 16
            s = adap_v[pl.ds(base, 16)][d % 16]
            spl_v[li * D + d, :] = lax.broadcast(s, (16,))

    isems = (is0, is1)
    gsems = (gs0, gs1)
    ssems = (ss0, ss1)

    def start_idx(c, buf):
        pltpu.async_copy(
            idxf.at[pl.ds((cbase + c) * NB, NB)], idx_v.at[buf], isems[buf])

    def wait_idx(buf):
        pltpu.make_async_copy(
            idxf.at[pl.ds(0, NB)], idx_v.at[buf], isems[buf]).wait()

    def compute_idx2(buf):
        for j in range(2):
            for k in range(8):
                x = idx_v[buf, pl.ds(j * 128 + k * 16, 16)]
                idx2_v[buf, j, pl.ds(k * 16, 16)] = lax.shift_left(x, one)

    def start_gather(buf):
        for j in range(2):
            pltpu.async_copy(
                table2.at[idx2_v.at[buf, j]],
                rows_v.at[buf, pl.ds(j * 128, 128), :],
                gsems[buf])

    def wait_gather(buf):
        for j in range(2):
            pltpu.make_async_copy(
                table2.at[idx2_v.at[buf, j]],
                rows_v.at[buf, pl.ds(j * 128, 128), :],
                gsems[buf]).wait()

    def add_chunk(c, buf):
        li = (cbase + c) // CPL - l0

        @plsc.parallel_loop(0, NB // 16)
        def _(blk):
            r0 = blk * 16
            row_ids = iota + r0
            for d in range(D):
                cols = jnp.full((16,), d, jnp.int32)
                vals = plsc.load_gather(rows_v.at[buf], [row_ids, cols])
                tr_v[buf, d, pl.ds(r0, 16)] = vals + spl_v[li * D + d, :]

    def start_store(c, buf):
        g = cbase + c
        pltpu.async_copy(
            tr_v.at[buf],
            out.at[g // CPL, :, pl.ds((g % CPL) * NB, NB)],
            ssems[buf])

    def wait_store(buf):
        pltpu.make_async_copy(
            tr_v.at[buf], out.at[0, :, pl.ds(0, NB)], ssems[buf]).wait()

    start_idx(0, 0)
    start_idx(1, 1)
    wait_idx(0)
    compute_idx2(0)
    start_gather(0)

    @pl.loop(0, CPW, step=2)
    def _(c):
        for b in range(2):
            cc = c + b
            buf = b
            obuf = 1 - b

            @pl.when(cc + 1 < CPW)
            def _():
                wait_idx(obuf)
                compute_idx2(obuf)

            wait_gather(buf)

            @pl.when(cc + 1 < CPW)
            def _():
                start_gather(obuf)

            @pl.when(cc >= 2)
            def _():
                wait_store(buf)

            add_chunk(cc, buf)
            start_store(cc, buf)

            @pl.when(cc + 2 < CPW)
            def _():
                start_idx(cc + 2, buf)

    wait_store(0)
    wait_store(1)


_sc_call = functools.partial(
    pl.kernel,
    out_type=jax.ShapeDtypeStruct((L, D, B), jnp.float32),
    mesh=plsc.VectorSubcoreMesh(
        core_axis_name="c", subcore_axis_name="s",
        num_cores=NC, num_subcores=NS),
    scratch_types=[
        pltpu.VMEM((2, NB), jnp.int32),
        pltpu.VMEM((2, 2, 128), jnp.int32),
        pltpu.VMEM((L * D,), jnp.float32),
        pltpu.VMEM((2 * D, 16), jnp.float32),
        pltpu.VMEM((2, NB, D), jnp.float32),
        pltpu.VMEM((2, D, NB), jnp.float32),
        pltpu.SemaphoreType.DMA,
        pltpu.SemaphoreType.DMA,
        pltpu.SemaphoreType.DMA,
        pltpu.SemaphoreType.DMA,
        pltpu.SemaphoreType.DMA,
        pltpu.SemaphoreType.DMA,
    ],
    compiler_params=pltpu.CompilerParams(
        needs_layout_passes=False, use_tc_tiling_on_sc=False),
)(_sc_body)


def kernel(indices, table, W_down, b_down, W_up, b_up):
    idx0 = indices[0]
    adap = _adapter_call(idx0, W_down, b_down.reshape(1, R),
                         W_up, b_up.reshape(1, D), table)
    out_ldb = _sc_call(_tp_call(table.T).reshape(2 * V, D),
                       indices.T.reshape(N),
                       adap.reshape(L * D))
    return out_ldb.transpose(2, 0, 1)


# speedup vs baseline: 1.3835x; 1.3835x over previous
import functools

import jax
import jax.numpy as jnp
from jax import lax
from jax.experimental import pallas as pl
from jax.experimental.pallas import tpu as pltpu
from jax.experimental.pallas import tpu_sc as plsc

V = 1000000
D = 64
R = 16
B = 16384
L = 20

NC, NS = 2, 16
NW = NC * NS
N = B * L
NB = 256
CPL = B // NB
CPW = L * CPL // NW


def _adap_body(idx0_ref, wd_ref, bd_ref, wu_ref, bu_ref, table_ref,
               out_ref, emb_ref, sem):
    for i in range(L):
        pltpu.make_async_copy(
            table_ref.at[pl.ds(idx0_ref[i], 1)], emb_ref.at[pl.ds(i, 1)], sem
        ).start()
    for i in range(L):
        pltpu.make_async_copy(
            table_ref.at[pl.ds(idx0_ref[i], 1)], emb_ref.at[pl.ds(i, 1)], sem
        ).wait()
    h = emb_ref[...]
    mid = jnp.maximum(
        jnp.dot(h, wd_ref[...], preferred_element_type=jnp.float32)
        + bd_ref[...], 0.0)
    out_ref[...] = (h
                    + jnp.dot(mid, wu_ref[...],
                              preferred_element_type=jnp.float32)
                    + bu_ref[...])


_adapter_call = pl.pallas_call(
    _adap_body,
    out_shape=jax.ShapeDtypeStruct((L, D), jnp.float32),
    in_specs=[
        pl.BlockSpec(memory_space=pltpu.SMEM),
        pl.BlockSpec(memory_space=pltpu.VMEM),
        pl.BlockSpec(memory_space=pltpu.VMEM),
        pl.BlockSpec(memory_space=pltpu.VMEM),
        pl.BlockSpec(memory_space=pltpu.VMEM),
        pl.BlockSpec(memory_space=pltpu.MemorySpace.HBM),
    ],
    out_specs=pl.BlockSpec(memory_space=pltpu.VMEM),
    scratch_shapes=[pltpu.VMEM((L, D), jnp.float32), pltpu.SemaphoreType.DMA],
)


def _sc_body(table2, idxf, adap, out,
             idx_v, idx2_v, adap_v, spl_v, rows_v, tr_v,
             is0, is1, is2, is3, gs0, gs1, gs2, gs3, ss0, ss1):
    wid = lax.axis_index("s") * NC + lax.axis_index("c")
    cbase = wid * CPW
    l0 = cbase // CPL

    pltpu.sync_copy(adap, adap_v)

    one = jnp.int32(1)
    iota = lax.iota(jnp.int32, 16)

    for li in range(2):
        l = jnp.minimum(l0 + li, L - 1)
        for d in range(D):
            base = l * D + (d // 16) * 16
            s = adap_v[pl.ds(base, 16)][d % 16]
            spl_v[li * D + d, :] = lax.broadcast(s, (16,))

    isems = (is0, is1, is2, is3)
    gsems = (gs0, gs1, gs2, gs3)
    ssems = (ss0, ss1)

    def start_idx(c, buf):
        pltpu.async_copy(
            idxf.at[pl.ds((cbase + c) * NB, NB)], idx_v.at[buf], isems[buf])

    def wait_idx(buf):
        pltpu.make_async_copy(
            idxf.at[pl.ds(0, NB)], idx_v.at[buf], isems[buf]).wait()

    def compute_idx2(buf):
        for j in range(2):
            for k in range(8):
                x = idx_v[buf, pl.ds(j * 128 + k * 16, 16)]
                idx2_v[buf, j, pl.ds(k * 16, 16)] = lax.shift_left(x, one)

    def start_gather(buf):
        for j in range(2):
            pltpu.async_copy(
                table2.at[idx2_v.at[buf, j]],
                rows_v.at[buf, pl.ds(j * 128, 128), :],
                gsems[buf])

    def wait_gather(buf):
        for j in range(2):
            pltpu.make_async_copy(
                table2.at[idx2_v.at[buf, j]],
                rows_v.at[buf, pl.ds(j * 128, 128), :],
                gsems[buf]).wait()

    def add_chunk(c, buf, tbuf):
        li = (cbase + c) // CPL - l0

        @plsc.parallel_loop(0, NB // 16)
        def _(blk):
            r0 = blk * 16
            row_ids = iota + r0
            for d in range(D):
                cols = jnp.full((16,), d, jnp.int32)
                vals = plsc.load_gather(rows_v.at[buf], [row_ids, cols])
                tr_v[tbuf, d, pl.ds(r0, 16)] = vals + spl_v[li * D + d, :]

    def start_store(c, tbuf):
        g = cbase + c
        pltpu.async_copy(
            tr_v.at[tbuf],
            out.at[g // CPL, :, pl.ds((g % CPL) * NB, NB)],
            ssems[tbuf])

    def wait_store(tbuf):
        pltpu.make_async_copy(
            tr_v.at[tbuf], out.at[0, :, pl.ds(0, NB)], ssems[tbuf]).wait()

    for p in range(4):
        start_idx(p, p)
    for p in range(3):
        wait_idx(p)
        compute_idx2(p)
        start_gather(p)

    @pl.loop(0, CPW, step=4)
    def _(c):
        for b in range(4):
            cc = c + b
            buf = b
            nbuf = (b + 3) % 4
            tbuf = b % 2

            @pl.when(cc + 3 < CPW)
            def _():
                wait_idx(nbuf)
                compute_idx2(nbuf)
                start_gather(nbuf)

            @pl.when(cc + 4 < CPW)
            def _():
                start_idx(cc + 4, buf)

            wait_gather(buf)

            @pl.when(cc >= 2)
            def _():
                wait_store(tbuf)

            add_chunk(cc, buf, tbuf)
            start_store(cc, tbuf)

    wait_store(0)
    wait_store(1)


_sc_call = functools.partial(
    pl.kernel,
    out_type=jax.ShapeDtypeStruct((L, D, B), jnp.float32),
    mesh=plsc.VectorSubcoreMesh(
        core_axis_name="c", subcore_axis_name="s",
        num_cores=NC, num_subcores=NS),
    scratch_types=[
        pltpu.VMEM((4, NB), jnp.int32),
        pltpu.VMEM((4, 2, 128), jnp.int32),
        pltpu.VMEM((L * D,), jnp.float32),
        pltpu.VMEM((2 * D, 16), jnp.float32),
        pltpu.VMEM((4, NB, D), jnp.float32),
        pltpu.VMEM((2, D, NB), jnp.float32),
        pltpu.SemaphoreType.DMA,
        pltpu.SemaphoreType.DMA,
        pltpu.SemaphoreType.DMA,
        pltpu.SemaphoreType.DMA,
        pltpu.SemaphoreType.DMA,
        pltpu.SemaphoreType.DMA,
        pltpu.SemaphoreType.DMA,
        pltpu.SemaphoreType.DMA,
        pltpu.SemaphoreType.DMA,
        pltpu.SemaphoreType.DMA,
    ],
    compiler_params=pltpu.CompilerParams(
        needs_layout_passes=False, use_tc_tiling_on_sc=False),
)(_sc_body)


def kernel(indices, table, W_down, b_down, W_up, b_up):
    idx0 = indices[0]
    adap = _adapter_call(idx0, W_down, b_down.reshape(1, R),
                         W_up, b_up.reshape(1, D), table)
    out_ldb = _sc_call(jnp.pad(table, ((0, 0), (0, D))).reshape(2 * V, D),
                       indices.T.reshape(N),
                       adap.reshape(L * D))
    return out_ldb.transpose(2, 0, 1)


# speedup vs baseline: 1.5331x; 1.1081x over previous
import functools

import jax
import jax.numpy as jnp
from jax import lax
from jax.experimental import pallas as pl
from jax.experimental.pallas import tpu as pltpu
from jax.experimental.pallas import tpu_sc as plsc

V = 1000000
D = 64
R = 16
B = 16384
L = 20

NC, NS = 2, 16
NW = NC * NS
N = B * L
NB = 256
CPL = B // NB
CPW = L * CPL // NW


def _adap_body(idx0_ref, wd_ref, bd_ref, wu_ref, bu_ref, table_ref,
               out_ref, emb_ref, sem):
    for i in range(L):
        pltpu.make_async_copy(
            table_ref.at[pl.ds(idx0_ref[i], 1)], emb_ref.at[pl.ds(i, 1)], sem
        ).start()
    for i in range(L):
        pltpu.make_async_copy(
            table_ref.at[pl.ds(idx0_ref[i], 1)], emb_ref.at[pl.ds(i, 1)], sem
        ).wait()
    h = emb_ref[...]
    mid = jnp.maximum(
        jnp.dot(h, wd_ref[...], preferred_element_type=jnp.float32)
        + bd_ref[...], 0.0)
    out_ref[...] = (h
                    + jnp.dot(mid, wu_ref[...],
                              preferred_element_type=jnp.float32)
                    + bu_ref[...])


_adapter_call = pl.pallas_call(
    _adap_body,
    out_shape=jax.ShapeDtypeStruct((L, D), jnp.float32),
    in_specs=[
        pl.BlockSpec(memory_space=pltpu.SMEM),
        pl.BlockSpec(memory_space=pltpu.VMEM),
        pl.BlockSpec(memory_space=pltpu.VMEM),
        pl.BlockSpec(memory_space=pltpu.VMEM),
        pl.BlockSpec(memory_space=pltpu.VMEM),
        pl.BlockSpec(memory_space=pltpu.MemorySpace.HBM),
    ],
    out_specs=pl.BlockSpec(memory_space=pltpu.VMEM),
    scratch_shapes=[pltpu.VMEM((L, D), jnp.float32), pltpu.SemaphoreType.DMA],
)


def _sc_body(table2, idxf, adap, out,
             idx_v, adap_v, spl_v, rows_v, tr_v,
             is0, is1, gs0, gs1, ss0, ss1):
    wid = lax.axis_index("s") * NC + lax.axis_index("c")
    cbase = wid * CPW
    l0 = cbase // CPL

    pltpu.sync_copy(adap, adap_v)

    iota = lax.iota(jnp.int32, 16)

    for li in range(2):
        l = jnp.minimum(l0 + li, L - 1)
        for d in range(D):
            base = l * D + (d // 16) * 16
            s = adap_v[pl.ds(base, 16)][d % 16]
            spl_v[li * D + d, :] = lax.broadcast(s, (16,))

    isems = (is0, is1)
    gsems = (gs0, gs1)
    ssems = (ss0, ss1)

    def start_idx(c, buf):
        pltpu.async_copy(
            idxf.at[pl.ds((cbase + c) * NB, NB)], idx_v.at[buf], isems[buf])

    def wait_idx(buf):
        pltpu.make_async_copy(
            idxf.at[pl.ds(0, NB)], idx_v.at[buf], isems[buf]).wait()

    def start_gather(buf):
        for j in range(2):
            pltpu.async_copy(
                table2.at[idx_v.at[buf, pl.ds(j * 128, 128)]],
                rows_v.at[buf, pl.ds(j * 128, 128), :],
                gsems[buf])

    def wait_gather(buf):
        for j in range(2):
            pltpu.make_async_copy(
                table2.at[idx_v.at[buf, pl.ds(j * 128, 128)]],
                rows_v.at[buf, pl.ds(j * 128, 128), :],
                gsems[buf]).wait()

    def add_chunk(c, buf):
        li = (cbase + c) // CPL - l0

        @plsc.parallel_loop(0, D)
        def _(d):
            spl = spl_v[li * D + d, :]
            cols = lax.broadcast(d, (16,))
            for blk in range(NB // 16):
                r0 = blk * 16
                vals = plsc.load_gather(rows_v.at[buf], [iota + r0, cols])
                tr_v[buf, d, pl.ds(r0, 16)] = vals + spl

    def start_store(c, buf):
        g = cbase + c
        pltpu.async_copy(
            tr_v.at[buf],
            out.at[g // CPL, :, pl.ds((g % CPL) * NB, NB)],
            ssems[buf])

    def wait_store(buf):
        pltpu.make_async_copy(
            tr_v.at[buf], out.at[0, :, pl.ds(0, NB)], ssems[buf]).wait()

    start_idx(0, 0)
    start_idx(1, 1)
    wait_idx(0)
    start_gather(0)

    @pl.loop(0, CPW, step=2)
    def _(c):
        for b in range(2):
            cc = c + b
            buf = b
            obuf = 1 - b

            @pl.when(cc + 1 < CPW)
            def _():
                wait_idx(obuf)

            wait_gather(buf)

            @pl.when(cc + 1 < CPW)
            def _():
                start_gather(obuf)

            @pl.when(cc >= 2)
            def _():
                wait_store(buf)

            add_chunk(cc, buf)
            start_store(cc, buf)

            @pl.when(cc + 2 < CPW)
            def _():
                start_idx(cc + 2, buf)

    wait_store(0)
    wait_store(1)


_sc_call = functools.partial(
    pl.kernel,
    out_type=jax.ShapeDtypeStruct((L, D, B), jnp.float32),
    mesh=plsc.VectorSubcoreMesh(
        core_axis_name="c", subcore_axis_name="s",
        num_cores=NC, num_subcores=NS),
    scratch_types=[
        pltpu.VMEM((2, NB), jnp.int32),
        pltpu.VMEM((L * D,), jnp.float32),
        pltpu.VMEM((2 * D, 16), jnp.float32),
        pltpu.VMEM((2, NB, 128), jnp.float32),
        pltpu.VMEM((2, D, NB), jnp.float32),
        pltpu.SemaphoreType.DMA,
        pltpu.SemaphoreType.DMA,
        pltpu.SemaphoreType.DMA,
        pltpu.SemaphoreType.DMA,
        pltpu.SemaphoreType.DMA,
        pltpu.SemaphoreType.DMA,
    ],
    compiler_params=pltpu.CompilerParams(needs_layout_passes=False),
)(_sc_body)


def kernel(indices, table, W_down, b_down, W_up, b_up):
    idx0 = indices[0]
    adap = _adapter_call(idx0, W_down, b_down.reshape(1, R),
                         W_up, b_up.reshape(1, D), table)
    out_ldb = _sc_call(jnp.pad(table, ((0, 0), (0, D))),
                       indices.T.reshape(N),
                       adap.reshape(L * D))
    return out_ldb.transpose(2, 0, 1)


# speedup vs baseline: 1.5355x; 1.0016x over previous
import functools

import jax
import jax.numpy as jnp
from jax import lax
from jax.experimental import pallas as pl
from jax.experimental.pallas import tpu as pltpu
from jax.experimental.pallas import tpu_sc as plsc

V = 1000000
D = 64
R = 16
B = 16384
L = 20

NC, NS = 2, 16
NW = NC * NS
N = B * L
NB = 256
CPL = B // NB
CPW = L * CPL // NW


def _adap_body(idx0_ref, wd_ref, bd_ref, wu_ref, bu_ref, table_ref,
               out_ref, emb_ref, sem):
    for i in range(L):
        pltpu.make_async_copy(
            table_ref.at[pl.ds(idx0_ref[i], 1)], emb_ref.at[pl.ds(i, 1)], sem
        ).start()
    for i in range(L):
        pltpu.make_async_copy(
            table_ref.at[pl.ds(idx0_ref[i], 1)], emb_ref.at[pl.ds(i, 1)], sem
        ).wait()
    h = emb_ref[...]
    mid = jnp.maximum(
        jnp.dot(h, wd_ref[...], preferred_element_type=jnp.float32)
        + bd_ref[...], 0.0)
    out_ref[...] = (h
                    + jnp.dot(mid, wu_ref[...],
                              preferred_element_type=jnp.float32)
                    + bu_ref[...])


_adapter_call = pl.pallas_call(
    _adap_body,
    out_shape=jax.ShapeDtypeStruct((L, D), jnp.float32),
    in_specs=[
        pl.BlockSpec(memory_space=pltpu.SMEM),
        pl.BlockSpec(memory_space=pltpu.VMEM),
        pl.BlockSpec(memory_space=pltpu.VMEM),
        pl.BlockSpec(memory_space=pltpu.VMEM),
        pl.BlockSpec(memory_space=pltpu.VMEM),
        pl.BlockSpec(memory_space=pltpu.MemorySpace.HBM),
    ],
    out_specs=pl.BlockSpec(memory_space=pltpu.VMEM),
    scratch_shapes=[pltpu.VMEM((L, D), jnp.float32), pltpu.SemaphoreType.DMA],
)


def _sc_body(table2, idxf, adap, out,
             idx_v, adap_v, spl_v, rows_v, tr_v,
             is0, is1, gs0, gs1, ss0, ss1):
    wid = lax.axis_index("s") * NC + lax.axis_index("c")
    cbase = wid * CPW
    l0 = cbase // CPL

    pltpu.sync_copy(adap, adap_v)

    iota = lax.iota(jnp.int32, 16)

    for li in range(2):
        l = jnp.minimum(l0 + li, L - 1)
        for d in range(D):
            base = l * D + (d // 16) * 16
            s = adap_v[pl.ds(base, 16)][d % 16]
            spl_v[li * D + d, :] = lax.broadcast(s, (16,))

    isems = (is0, is1)
    gsems = (gs0, gs1)
    ssems = (ss0, ss1)

    def start_idx(c, buf):
        pltpu.async_copy(
            idxf.at[pl.ds((cbase + c) * NB, NB)], idx_v.at[buf], isems[buf])

    def wait_idx(buf):
        pltpu.make_async_copy(
            idxf.at[pl.ds(0, NB)], idx_v.at[buf], isems[buf]).wait()

    def start_gather(buf):
        for j in range(2):
            pltpu.async_copy(
                table2.at[idx_v.at[buf, pl.ds(j * 128, 128)]],
                rows_v.at[buf, pl.ds(j * 128, 128), :],
                gsems[buf])

    def wait_gather(buf):
        for j in range(2):
            pltpu.make_async_copy(
                table2.at[idx_v.at[buf, pl.ds(j * 128, 128)]],
                rows_v.at[buf, pl.ds(j * 128, 128), :],
                gsems[buf]).wait()

    def add_chunk(c, buf):
        li = (cbase + c) // CPL - l0

        @plsc.parallel_loop(0, D, unroll=4)
        def _(d):
            spl = spl_v[li * D + d, :]
            cols = lax.broadcast(d, (16,))
            for blk in range(NB // 16):
                r0 = blk * 16
                vals = plsc.load_gather(rows_v.at[buf], [iota + r0, cols])
                tr_v[buf, d, pl.ds(r0, 16)] = vals + spl

    def start_store(c, buf):
        g = cbase + c
        pltpu.async_copy(
            tr_v.at[buf],
            out.at[g // CPL, :, pl.ds((g % CPL) * NB, NB)],
            ssems[buf])

    def wait_store(buf):
        pltpu.make_async_copy(
            tr_v.at[buf], out.at[0, :, pl.ds(0, NB)], ssems[buf]).wait()

    start_idx(0, 0)
    start_idx(1, 1)
    wait_idx(0)
    start_gather(0)

    @pl.loop(0, CPW, step=2)
    def _(c):
        for b in range(2):
            cc = c + b
            buf = b
            obuf = 1 - b

            @pl.when(cc + 1 < CPW)
            def _():
                wait_idx(obuf)

            wait_gather(buf)

            @pl.when(cc + 1 < CPW)
            def _():
                start_gather(obuf)

            @pl.when(cc >= 2)
            def _():
                wait_store(buf)

            add_chunk(cc, buf)
            start_store(cc, buf)

            @pl.when(cc + 2 < CPW)
            def _():
                start_idx(cc + 2, buf)

    wait_store(0)
    wait_store(1)


_sc_call = functools.partial(
    pl.kernel,
    out_type=jax.ShapeDtypeStruct((L, D, B), jnp.float32),
    mesh=plsc.VectorSubcoreMesh(
        core_axis_name="c", subcore_axis_name="s",
        num_cores=NC, num_subcores=NS),
    scratch_types=[
        pltpu.VMEM((2, NB), jnp.int32),
        pltpu.VMEM((L * D,), jnp.float32),
        pltpu.VMEM((2 * D, 16), jnp.float32),
        pltpu.VMEM((2, NB, 128), jnp.float32),
        pltpu.VMEM((2, D, NB), jnp.float32),
        pltpu.SemaphoreType.DMA,
        pltpu.SemaphoreType.DMA,
        pltpu.SemaphoreType.DMA,
        pltpu.SemaphoreType.DMA,
        pltpu.SemaphoreType.DMA,
        pltpu.SemaphoreType.DMA,
    ],
    compiler_params=pltpu.CompilerParams(needs_layout_passes=False),
)(_sc_body)


def kernel(indices, table, W_down, b_down, W_up, b_up):
    idx0 = indices[0]
    adap = _adapter_call(idx0, W_down, b_down.reshape(1, R),
                         W_up, b_up.reshape(1, D), table)
    out_ldb = _sc_call(jnp.pad(table, ((0, 0), (0, D))),
                       indices.T.reshape(N),
                       adap.reshape(L * D))
    return out_ldb.transpose(2, 0, 1)
